# Initial kernel scaffold; baseline (speedup 1.0000x reference)
import functools

import jax
import jax.numpy as jnp
from jax import lax
from jax.experimental import pallas as pl
from jax.experimental.pallas import tpu as pltpu
from jax.experimental.pallas import tpu_sc as plsc

B, L = 1024, 200
N = B * L
D = 64
HD = 192
NF = 6

NC, NS = 2, 16
NW = NC * NS
TPW = N // NW
CH = 128
NCHUNK = TPW // CH

BT = 1024


def _sc_gather_body(*refs):
    tables = refs[0:NF]
    idxs = refs[NF:2 * NF]
    outs = refs[2 * NF:3 * NF]
    ivs = refs[3 * NF:4 * NF]
    rbufs = refs[4 * NF:5 * NF]
    sem = refs[5 * NF]

    wid = lax.axis_index("s") * NC + lax.axis_index("c")
    base = wid * TPW

    for f in range(NF):
        pltpu.sync_copy(idxs[f].at[wid], ivs[f])

    def chunk_body(c, carry):
        handles = []
        for f in range(NF):
            handles.append(
                pltpu.async_copy(tables[f].at[ivs[f].at[c]], rbufs[f], sem))
        for h in handles:
            h.wait()
        off = base + c * CH
        for f in range(NF):
            pltpu.sync_copy(rbufs[f], outs[f].at[pl.ds(off, CH)])
        return carry

    lax.fori_loop(0, NCHUNK, chunk_body, 0)


_sc_gather = pl.kernel(
    _sc_gather_body,
    out_type=tuple(jax.ShapeDtypeStruct((N, D), jnp.float32) for _ in range(NF)),
    mesh=plsc.VectorSubcoreMesh(
        core_axis_name="c", subcore_axis_name="s",
        num_cores=NC, num_subcores=NS),
    scratch_types=(
        [pltpu.VMEM((NCHUNK, CH), jnp.int32) for _ in range(NF)]
        + [pltpu.VMEM((CH, D), jnp.float32) for _ in range(NF)]
        + [pltpu.SemaphoreType.DMA]
    ),
)


def _tc_proj_body(e0, e1, e2, e3, e4, e5, wc, bc, out):
    e = jnp.concatenate(
        [e0[...], e1[...], e2[...], e3[...], e4[...], e5[...]], axis=1)
    out[...] = jnp.dot(e, wc[...],
                       preferred_element_type=jnp.float32) + bc[...]


@jax.jit
def _run(tables, idxs, W_comb, b_comb):
    es = _sc_gather(*tables, *idxs)
    grid = N // BT
    espec = pl.BlockSpec((BT, D), lambda i: (i, 0))
    x = pl.pallas_call(
        _tc_proj_body,
        grid=(grid,),
        in_specs=[espec] * NF + [
            pl.BlockSpec((NF * D, HD), lambda i: (0, 0)),
            pl.BlockSpec((1, HD), lambda i: (0, 0)),
        ],
        out_specs=pl.BlockSpec((BT, HD), lambda i: (i, 0)),
        out_shape=jax.ShapeDtypeStruct((N, HD), jnp.float32),
    )(*es, W_comb, b_comb.reshape(1, HD))
    return x.reshape(B, L, HD)


def kernel(correct, question, test, tag, elapsed_question, elapsed_test,
           mask, interaction, index,
           W_interaction, W_question, W_test, W_tag, W_elapsed_question,
           W_elapsed_test, W_comb, b_comb):
    idxs = tuple(a.reshape(NW, NCHUNK, CH) for a in (
        interaction, question, test, tag, elapsed_question, elapsed_test))
    tables = (W_interaction, W_question, W_test, W_tag,
              W_elapsed_question, W_test)
    return _run(tables, idxs, W_comb, b_comb)


# speedup vs baseline: 1.1590x; 1.1590x over previous
import functools

import jax
import jax.numpy as jnp
from jax import lax
from jax.experimental import pallas as pl
from jax.experimental.pallas import tpu as pltpu
from jax.experimental.pallas import tpu_sc as plsc

B, L = 1024, 200
N = B * L
D = 64
HD = 192
NF = 6

NC, NS = 2, 16
NW = NC * NS
TPW = N // NW
CH = 128
NCHUNK = TPW // CH

BT = 1024


def _sc_gather_body(*refs):
    tables = refs[0:NF]
    idxs = refs[NF:2 * NF]
    outs = refs[2 * NF:3 * NF]
    ivs = refs[3 * NF:4 * NF]
    rbufs = refs[4 * NF:5 * NF]
    sem = refs[5 * NF]

    wid = lax.axis_index("s") * NC + lax.axis_index("c")
    base = wid * TPW

    for f in range(NF):
        pltpu.sync_copy(idxs[f].at[wid], ivs[f])

    def chunk_body(c, carry):
        handles = []
        for f in range(NF):
            handles.append(
                pltpu.async_copy(tables[f].at[ivs[f].at[c]], rbufs[f], sem))
        for h in handles:
            h.wait()
        off = base + c * CH
        for f in range(NF):
            pltpu.sync_copy(rbufs[f], outs[f].at[pl.ds(off, CH)])
        return carry

    lax.fori_loop(0, NCHUNK, chunk_body, 0)


_sc_gather = pl.kernel(
    _sc_gather_body,
    out_type=tuple(jax.ShapeDtypeStruct((N, D), jnp.float32) for _ in range(NF)),
    mesh=plsc.VectorSubcoreMesh(
        core_axis_name="c", subcore_axis_name="s",
        num_cores=NC, num_subcores=NS),
    scratch_types=(
        [pltpu.VMEM((NCHUNK, CH), jnp.int32) for _ in range(NF)]
        + [pltpu.VMEM((CH, D), jnp.float32) for _ in range(NF)]
        + [pltpu.SemaphoreType.DMA]
    ),
    compiler_params=pltpu.CompilerParams(use_tc_tiling_on_sc=False),
)


def _tc_proj_body(e0, e1, e2, e3, e4, e5, wc, bc, out):
    e = jnp.concatenate(
        [e0[...], e1[...], e2[...], e3[...], e4[...], e5[...]], axis=1)
    out[...] = jnp.dot(e, wc[...],
                       preferred_element_type=jnp.float32) + bc[...]


@jax.jit
def _run(tables, idxs, W_comb, b_comb):
    es = _sc_gather(*tables, *idxs)
    grid = N // BT
    espec = pl.BlockSpec((BT, D), lambda i: (i, 0))
    x = pl.pallas_call(
        _tc_proj_body,
        grid=(grid,),
        in_specs=[espec] * NF + [
            pl.BlockSpec((NF * D, HD), lambda i: (0, 0)),
            pl.BlockSpec((1, HD), lambda i: (0, 0)),
        ],
        out_specs=pl.BlockSpec((BT, HD), lambda i: (i, 0)),
        out_shape=jax.ShapeDtypeStruct((N, HD), jnp.float32),
    )(*es, W_comb, b_comb.reshape(1, HD))
    return x.reshape(B, L, HD)


def kernel(correct, question, test, tag, elapsed_question, elapsed_test,
           mask, interaction, index,
           W_interaction, W_question, W_test, W_tag, W_elapsed_question,
           W_elapsed_test, W_comb, b_comb):
    idxs = tuple(a.reshape(NW, NCHUNK, CH) for a in (
        interaction, question, test, tag, elapsed_question, elapsed_test))
    tables = (W_interaction, W_question, W_test, W_tag,
              W_elapsed_question, W_test)
    return _run(tables, idxs, W_comb, b_comb)
